# baseline (speedup 1.0000x reference)
import functools

import jax
import jax.numpy as jnp
from jax import lax
from jax.experimental import pallas as pl
from jax.experimental.pallas import tpu as pltpu
from jax.experimental.pallas import tpu_sc as plsc

_E = 8
_H = 2048
_I = 2048
_T = 64
_JITTER = 0.01
_BI = 512
_NB = _I // _BI
_L = 16
_NG = _T // _L


def _nt_dot(a, b):
    return jax.lax.dot_general(a, b, (((1,), (1,)), ((), ())),
                               preferred_element_type=jnp.float32)


def _logits_body(x_ref, g_ref, o_ref):
    o_ref[...] = _nt_dot(g_ref[...], x_ref[...])


def _sc_route_body(logits_hbm, out_hbm, lv, ov):
    wid = lax.axis_index("s") * 2 + lax.axis_index("c")

    @pl.when(wid == 0)
    def _():
        pltpu.sync_copy(logits_hbm, lv)
        thr = jnp.float32(2.0 * _JITTER)
        for g in range(_NG):
            sl = pl.ds(g * _L, _L)
            s = [lv[e, sl] for e in range(_E)]
            mlt = s[0]
            for e in range(1, _E):
                mlt = jnp.maximum(mlt, s[e])
            ind1 = jnp.full((_L,), _E - 1, jnp.int32)
            for e in range(_E - 2, -1, -1):
                ind1 = jnp.where(s[e] == mlt, jnp.int32(e), ind1)
            sum1 = jnp.zeros((_L,), jnp.float32)
            for e in range(_E):
                factor = jnp.maximum(jnp.abs(s[e]), mlt)
                masked = (mlt - s[e]) > thr * factor
                sum1 += jnp.where(masked, 0.0, jnp.exp(s[e] - mlt))
            m1 = 1.0 / sum1
            ms = [jnp.where(ind1 == e, jnp.float32(-1e30), s[e])
                  for e in range(_E)]
            mlt2 = ms[0]
            for e in range(1, _E):
                mlt2 = jnp.maximum(mlt2, ms[e])
            ind2 = jnp.full((_L,), _E - 1, jnp.int32)
            for e in range(_E - 2, -1, -1):
                ind2 = jnp.where(ms[e] == mlt2, jnp.int32(e), ind2)
            sum2 = jnp.zeros((_L,), jnp.float32)
            for e in range(_E):
                factor2 = jnp.maximum(jnp.abs(s[e]), mlt2)
                masked2 = ((mlt2 - s[e]) > thr * factor2) | (ind1 == e)
                sum2 += jnp.where(masked2, 0.0, jnp.exp(s[e] - mlt2))
            m2 = 1.0 / sum2
            for e in range(_E):
                r = (jnp.where(ind1 == e, m1, 0.0)
                     + jnp.where(ind2 == e, m2, 0.0))
                ov[e, sl] = r
        pltpu.sync_copy(ov, out_hbm)


def _moe_body(x_ref, rt_ref, w1_ref, w3_ref, w2_ref, out_ref):
    e = pl.program_id(0)
    i = pl.program_id(1)

    @pl.when((e == 0) & (i == 0))
    def _init():
        out_ref[...] = jnp.zeros_like(out_ref)

    x = x_ref[...]
    h1 = _nt_dot(x, w1_ref[0, 0])
    h3 = _nt_dot(x, w3_ref[0, 0])
    act = h1 * jax.nn.sigmoid(h1) * h3
    oh = (jax.lax.broadcasted_iota(jnp.int32, (1, _E), 1) == e
          ).astype(jnp.float32)
    scale = jax.lax.dot_general(rt_ref[...], oh, (((0,), (1,)), ((), ())),
                                preferred_element_type=jnp.float32)
    out_ref[...] += _nt_dot(scale * act, w2_ref[0])


def kernel(hidden_states, gate_w, ws, w2s):
    logits = pl.pallas_call(
        _logits_body,
        out_shape=jax.ShapeDtypeStruct((_E, _T), jnp.float32),
    )(hidden_states, gate_w)

    mesh = plsc.VectorSubcoreMesh(core_axis_name="c", subcore_axis_name="s")
    routing = functools.partial(
        pl.kernel,
        mesh=mesh,
        out_type=jax.ShapeDtypeStruct((_E, _T), jnp.float32),
        scratch_types=[
            pltpu.VMEM((_E, _T), jnp.float32),
            pltpu.VMEM((_E, _T), jnp.float32),
        ],
    )(_sc_route_body)(logits)

    ws4 = ws.reshape(_E, 2, _I, _H)
    grid = (_E, _NB)
    return pl.pallas_call(
        _moe_body,
        grid=grid,
        in_specs=[
            pl.BlockSpec((_T, _H), lambda e, i: (0, 0)),
            pl.BlockSpec((_E, _T), lambda e, i: (0, 0)),
            pl.BlockSpec((1, 1, _BI, _H), lambda e, i: (e, 0, i, 0)),
            pl.BlockSpec((1, 1, _BI, _H), lambda e, i: (e, 1, i, 0)),
            pl.BlockSpec((1, _H, _BI), lambda e, i: (e, 0, i)),
        ],
        out_specs=pl.BlockSpec((_T, _H), lambda e, i: (0, 0)),
        out_shape=jax.ShapeDtypeStruct((_T, _H), jnp.float32),
    )(hidden_states, routing, ws4, ws4, w2s)


# speedup vs baseline: 1.1672x; 1.1672x over previous
import jax
import jax.numpy as jnp
from jax.experimental import pallas as pl
from jax.experimental.pallas import tpu as pltpu

_E = 8
_H = 2048
_I = 2048
_T = 64
_JITTER = 0.01
_BI = 512
_NB = _I // _BI


def _sparsemixer_routing(scores):
    lanes = jax.lax.broadcasted_iota(jnp.int32, scores.shape, 1)
    neg_inf = jnp.float32(-jnp.inf)
    mlt = jnp.max(scores, axis=-1, keepdims=True)
    eq1 = scores == mlt
    ind1 = jnp.min(jnp.where(eq1, lanes, _E), axis=-1, keepdims=True)
    oh1 = lanes == ind1
    factor = jnp.maximum(jnp.abs(scores), mlt)
    mask1 = (mlt - scores) / factor > 2.0 * _JITTER
    mg1 = jnp.where(mask1, neg_inf, scores)
    sm1 = jax.nn.softmax(mg1, axis=-1)
    m1 = jnp.sum(jnp.where(oh1, sm1, 0.0), axis=-1, keepdims=True)
    masked_scores = jnp.where(oh1, neg_inf, scores)
    mlt2 = jnp.max(masked_scores, axis=-1, keepdims=True)
    eq2 = masked_scores == mlt2
    ind2 = jnp.min(jnp.where(eq2, lanes, _E), axis=-1, keepdims=True)
    oh2 = lanes == ind2
    factor2 = jnp.maximum(jnp.abs(scores), mlt2)
    mask2 = (mlt2 - scores) / factor2 > 2.0 * _JITTER
    mg2 = jnp.where(mask2, neg_inf, masked_scores)
    sm2 = jax.nn.softmax(mg2, axis=-1)
    m2 = jnp.sum(jnp.where(oh2, sm2, 0.0), axis=-1, keepdims=True)
    return jnp.where(oh1, m1, 0.0) + jnp.where(oh2, m2, 0.0)


def _nt_dot(a, b):
    return jax.lax.dot_general(a, b, (((1,), (1,)), ((), ())),
                               preferred_element_type=jnp.float32)


def _moe_body(x_ref, gate_ref, w13_ref, w2_ref, out_ref, rt_ref):
    e = pl.program_id(0)
    i = pl.program_id(1)

    @pl.when((e == 0) & (i == 0))
    def _init():
        logits = _nt_dot(x_ref[...], gate_ref[...])
        rt_ref[...] = _sparsemixer_routing(logits)
        out_ref[...] = jnp.zeros_like(out_ref)

    x = x_ref[...]
    w13 = w13_ref[0, :, 0].reshape(2 * _BI, _H)
    h = _nt_dot(x, w13)
    h1 = h[:, :_BI]
    h3 = h[:, _BI:]
    act = h1 * jax.nn.sigmoid(h1) * h3
    lanes = jax.lax.broadcasted_iota(jnp.int32, (_T, _E), 1)
    scale = jnp.sum(jnp.where(lanes == e, rt_ref[...], 0.0),
                    axis=-1, keepdims=True)
    out_ref[...] += _nt_dot(scale * act, w2_ref[0])


def kernel(hidden_states, gate_w, ws, w2s):
    ws5 = ws.reshape(_E, 2, _NB, _BI, _H)
    grid = (_E, _NB)
    return pl.pallas_call(
        _moe_body,
        grid=grid,
        in_specs=[
            pl.BlockSpec((_T, _H), lambda e, i: (0, 0)),
            pl.BlockSpec((_E, _H), lambda e, i: (0, 0)),
            pl.BlockSpec((1, 2, 1, _BI, _H), lambda e, i: (e, 0, i, 0, 0)),
            pl.BlockSpec((1, _H, _BI), lambda e, i: (e, 0, i)),
        ],
        out_specs=pl.BlockSpec((_T, _H), lambda e, i: (0, 0)),
        out_shape=jax.ShapeDtypeStruct((_T, _H), jnp.float32),
        scratch_shapes=[pltpu.VMEM((_T, _E), jnp.float32)],
    )(hidden_states, gate_w, ws5, w2s)
